# Initial kernel scaffold; baseline (speedup 1.0000x reference)
import jax
import jax.numpy as jnp
from jax.experimental import pallas as pl


def kernel(x, table, W, b):
    raise NotImplementedError("write your pallas kernel here")


# speedup vs baseline: 1.1166x; 1.1166x over previous
import functools

import jax
import jax.numpy as jnp
from jax import lax
from jax.experimental import pallas as pl
from jax.experimental.pallas import tpu as pltpu
from jax.experimental.pallas import tpu_sc as plsc

_BATCH = 16384
_EMBED = 16
_NW = 32
_NPW = _BATCH // _NW
_NCHUNK = 4
_CSZ = _NPW // _NCHUNK
_NGRP = _NPW // 16


def _body(x_hbm, table_hbm, wp_hbm, out_hbm,
          x_v, wp_v, il_v, ic_v, rows_l, rows_c, acc_v, out_v, sem):
    wid = lax.axis_index("s") * 2 + lax.axis_index("c")
    xbase = wid * (_NPW * 6)

    pltpu.sync_copy(x_hbm.at[pl.ds(xbase, _NPW * 6)], x_v)
    pltpu.sync_copy(wp_hbm, wp_v)

    iota = lax.iota(jnp.int32, 16)
    iota6 = iota * 6

    wlbl = wp_v[pl.ds(32 * 16, 16)]
    wdb = wp_v[pl.ds(33 * 16, 16)]

    for g in range(_NCHUNK):
        for t in range(_CSZ // 16):
            off = (g * _CSZ + t * 16) * 6 + iota6
            ul = plsc.load_gather(x_v, [off])
            lb = plsc.load_gather(x_v, [off + 2])
            uc = plsc.load_gather(x_v, [off + 3])
            il_v[g, pl.ds(t * 16, 16)] = ul.astype(jnp.int32)
            ic_v[g, pl.ds(t * 16, 16)] = uc.astype(jnp.int32)
            acc_v[pl.ds(g * _CSZ + t * 16, 16)] = lb * wlbl + wdb

    copies = []
    for g in range(_NCHUNK):
        copies.append(pltpu.async_copy(
            table_hbm.at[il_v.at[g]], rows_l.at[pl.ds(g * _CSZ, _CSZ)], sem))
        copies.append(pltpu.async_copy(
            table_hbm.at[ic_v.at[g]], rows_c.at[pl.ds(g * _CSZ, _CSZ)], sem))
    for c in copies:
        c.wait()

    wvl = [wp_v[pl.ds(k * 16, 16)] for k in range(16)]
    wvc = [wp_v[pl.ds((16 + k) * 16, 16)] for k in range(16)]
    cols = [(iota + k) & 15 for k in range(16)]
    zeros = iota * 0
    ones = zeros + 1

    def p3(g, carry):
        for t in range(_CSZ // 16):
            row0 = g * _CSZ + t * 16
            rowv = row0 + iota
            acc = acc_v[pl.ds(row0, 16)]
            for k in range(16):
                vl = plsc.load_gather(rows_l, [rowv, cols[k]])
                acc = acc + vl * wvl[k]
            for k in range(16):
                vc = plsc.load_gather(rows_c, [rowv, cols[k]])
                acc = acc + vc * wvc[k]
            e0 = jnp.exp(jnp.minimum(-acc, 0.0))
            e1 = jnp.exp(jnp.minimum(acc, 0.0))
            rz = 1.0 / (e0 + e1)
            plsc.store_scatter(out_v, [rowv, zeros], e0 * rz)
            plsc.store_scatter(out_v, [rowv, ones], e1 * rz)
        return carry

    lax.fori_loop(0, _NCHUNK, p3, 0)

    pltpu.sync_copy(out_v, out_hbm.at[pl.ds(wid * _NPW, _NPW)])


@functools.partial(jax.jit, static_argnums=())
def _run(x_flat, table, wp):
    mesh = plsc.VectorSubcoreMesh(core_axis_name="c", subcore_axis_name="s")
    f = pl.kernel(
        _body,
        out_type=jax.ShapeDtypeStruct((_BATCH, 2), jnp.float32),
        mesh=mesh,
        scratch_types=[
            pltpu.VMEM((_NPW * 6,), jnp.float32),
            pltpu.VMEM((34 * 16,), jnp.float32),
            pltpu.VMEM((_NCHUNK, _CSZ), jnp.int32),
            pltpu.VMEM((_NCHUNK, _CSZ), jnp.int32),
            pltpu.VMEM((_NPW, _EMBED), jnp.float32),
            pltpu.VMEM((_NPW, _EMBED), jnp.float32),
            pltpu.VMEM((_NPW,), jnp.float32),
            pltpu.VMEM((_NPW, 2), jnp.float32),
            pltpu.SemaphoreType.DMA,
        ],
        compiler_params=pltpu.CompilerParams(
            needs_layout_passes=False, use_tc_tiling_on_sc=False),
    )
    return f(x_flat, table, wp)


def kernel(x, table, W, b):
    wd = W[1] - W[0]
    i = jnp.arange(16)
    perm = (i[None, :] + i[:, None]) % 16
    wpa = wd[0:16][perm]
    wpb = wd[16:32][perm]
    wlbl = jnp.full((1, 16), wd[32], jnp.float32)
    wdb = jnp.full((1, 16), b[1] - b[0], jnp.float32)
    wp = jnp.concatenate([wpa, wpb, wlbl, wdb], axis=0).reshape(-1)
    return _run(x.reshape(-1), table, wp)
